# baseline (speedup 1.0000x reference)
import functools

import jax
import jax.numpy as jnp
from jax import lax
from jax.experimental import pallas as pl
from jax.experimental.pallas import tpu as pltpu
from jax.experimental.pallas import tpu_sc as plsc

_N = 10000
_E = 320000
_NS = 16
_CHUNK = 128
_TCH = 160
_PAIRS = _TCH // 2
_TEDGE = _TCH * _CHUNK
_EPAD = _NS * _TEDGE
_AGGR = _N + 16
_STRIPE = 624
_ZR = 64
_ROWB = 1000
_NBLK = 2 * _N // _ROWB
_BPG = _NBLK // 2


def _make_segsum(nh, interpret=False):
    mesh = plsc.VectorSubcoreMesh(core_axis_name="c", subcore_axis_name="s",
                                  num_cores=2, num_subcores=_NS)
    out_type = [jax.ShapeDtypeStruct((2 * _N, 128), jnp.float32)
                for _ in range(nh)]
    scratch = [
        pltpu.VMEM_SHARED((_AGGR, 128), jnp.float32),
        pltpu.VMEM((_ZR, 128), jnp.float32),
        pltpu.VMEM((_CHUNK,), jnp.int32),
        pltpu.VMEM((_CHUNK,), jnp.int32),
        pltpu.VMEM((_CHUNK,), jnp.int32),
        pltpu.VMEM((_CHUNK,), jnp.int32),
        pltpu.VMEM((_CHUNK, 128), jnp.float32),
        pltpu.VMEM((_CHUNK, 128), jnp.float32),
        pltpu.SemaphoreType.DMA,
        pltpu.SemaphoreType.DMA,
    ]

    @functools.partial(pl.kernel, mesh=mesh, out_type=out_type,
                       scratch_types=scratch, interpret=interpret)
    def segsum(srcs, dsts, *rest):
        xtabs = rest[:nh]
        outs = rest[nh:2 * nh]
        (agg, zrow, src0, src1, dst0, dst1, rows0, rows1,
         sem0, sem1) = rest[2 * nh:]
        c = lax.axis_index("c")
        s = lax.axis_index("s")
        ebase = c * _EPAD + s * _TEDGE

        def _zz(i, carry):
            zrow[i // 8, pl.ds((i % 8) * 16, 16)] = jnp.zeros((16,),
                                                              jnp.float32)
            return carry
        lax.fori_loop(0, _ZR * 8, _zz, 0)

        for h in range(nh):
            xt = xtabs[h]
            for q in range(640 // _ZR):
                pltpu.sync_copy(zrow,
                                agg.at[pl.ds(s * _STRIPE + q * _ZR, _ZR)])

            @pl.when(s == _NS - 1)
            def _():
                pltpu.sync_copy(zrow.at[pl.ds(0, _AGGR - _N)],
                                agg.at[pl.ds(_N, _AGGR - _N)])
            plsc.subcore_barrier()

            pltpu.sync_copy(srcs.at[pl.ds(ebase, _CHUNK)], src0)
            pltpu.sync_copy(dsts.at[pl.ds(ebase, _CHUNK)], dst0)
            pltpu.async_copy(xt.at[src0], rows0, sem0)
            pltpu.sync_copy(srcs.at[pl.ds(ebase + _CHUNK, _CHUNK)], src1)
            pltpu.sync_copy(dsts.at[pl.ds(ebase + _CHUNK, _CHUNK)], dst1)
            pltpu.async_copy(xt.at[src1], rows1, sem1)

            def _pair(j, carry):
                i0 = 2 * j
                pltpu.make_async_copy(xt.at[pl.ds(0, _CHUNK)], rows0,
                                      sem0).wait()
                pltpu.sync_copy(rows0, agg.at[dst0], add=True)

                @pl.when(j < _PAIRS - 1)
                def _():
                    nb = ebase + (i0 + 2) * _CHUNK
                    pltpu.sync_copy(srcs.at[pl.ds(nb, _CHUNK)], src0)
                    pltpu.sync_copy(dsts.at[pl.ds(nb, _CHUNK)], dst0)
                    pltpu.async_copy(xt.at[src0], rows0, sem0)
                pltpu.make_async_copy(xt.at[pl.ds(0, _CHUNK)], rows1,
                                      sem1).wait()
                pltpu.sync_copy(rows1, agg.at[dst1], add=True)

                @pl.when(j < _PAIRS - 1)
                def _():
                    nb = ebase + (i0 + 3) * _CHUNK
                    pltpu.sync_copy(srcs.at[pl.ds(nb, _CHUNK)], src1)
                    pltpu.sync_copy(dsts.at[pl.ds(nb, _CHUNK)], dst1)
                    pltpu.async_copy(xt.at[src1], rows1, sem1)
                return carry
            lax.fori_loop(0, _PAIRS, _pair, 0)
            plsc.subcore_barrier()
            pltpu.sync_copy(agg.at[pl.ds(s * _STRIPE, _STRIPE)],
                            outs[h].at[pl.ds(c * _N + s * _STRIPE, _STRIPE)])

            @pl.when(s == _NS - 1)
            def _():
                pltpu.sync_copy(
                    agg.at[pl.ds(_NS * _STRIPE, _N - _NS * _STRIPE)],
                    outs[h].at[pl.ds(c * _N + _NS * _STRIPE,
                                     _N - _NS * _STRIPE)])

    return segsum


_segsum_cache = {}


def _segsum(nh):
    if nh not in _segsum_cache:
        _segsum_cache[nh] = _make_segsum(nh)
    return _segsum_cache[nh]


def _full(shape):
    return pl.BlockSpec(shape, lambda *a: tuple(0 for _ in shape))


def _make_gin_mm(nh, din, dout, interpret=False):
    def body(*refs):
        eps_ref = refs[0]
        x_refs = refs[1:1 + nh]
        agg_refs = refs[1 + nh:1 + 2 * nh]
        w1, b1, w2, b2 = refs[1 + 2 * nh:5 + 2 * nh]
        t_ref, mom = refs[5 + 2 * nh:7 + 2 * nh]
        b = pl.program_id(0)

        eps1 = 1.0 + eps_ref[0, 0]
        parts = [eps1 * x_refs[i][...] + agg_refs[i][...] for i in range(nh)]
        h = parts[0] if nh == 1 else jnp.concatenate(parts, axis=1)
        h1 = jnp.maximum(
            jnp.dot(h, w1[...], preferred_element_type=jnp.float32) + b1[...],
            0.0)
        t = jnp.dot(h1, w2[...], preferred_element_type=jnp.float32) + b2[...]
        t_ref[...] = t

        @pl.when(b % _BPG == 0)
        def _():
            mom[...] = jnp.zeros_like(mom)
        mom[0, 0, :] += jnp.sum(t, axis=0)
        mom[0, 1, :] += jnp.sum(t * t, axis=0)

    hw = 128 if nh > 1 else din
    in_specs = [_full((1, 1))]
    in_specs += [pl.BlockSpec((_ROWB, hw), lambda b: (b, 0))
                 for _ in range(2 * nh)]
    in_specs += [_full((din, dout)), _full((1, dout)),
                 _full((dout, dout)), _full((1, dout))]
    out_specs = [pl.BlockSpec((_ROWB, dout), lambda b: (b, 0)),
                 pl.BlockSpec((1, 2, dout), lambda b: (b // _BPG, 0, 0))]
    out_shape = [jax.ShapeDtypeStruct((2 * _N, dout), jnp.float32),
                 jax.ShapeDtypeStruct((2, 2, dout), jnp.float32)]
    return pl.pallas_call(body, grid=(_NBLK,), in_specs=in_specs,
                          out_specs=out_specs, out_shape=out_shape,
                          interpret=interpret)


def _make_norm(dout, relu, nout, colsum, interpret=False):
    def body(t_ref, mom_ref, g_ref, b_ref, *o_refs):
        b = pl.program_id(0)
        mom = mom_ref[...]
        mu = mom[0, 0:1, :] / (1.0 * _N)
        sq = mom[0, 1:2, :] / (1.0 * _N)
        var = sq - mu * mu
        xn = (t_ref[...] - mu) * lax.rsqrt(var + 1e-5) * g_ref[...] + b_ref[...]
        if relu:
            xn = jnp.maximum(xn, 0.0)
        w = dout // nout
        for i in range(nout):
            o_refs[i][...] = xn[:, i * w:(i + 1) * w]
        if colsum:
            cs = o_refs[nout]

            @pl.when(b % _BPG == 0)
            def _():
                cs[...] = jnp.zeros_like(cs)
            cs[0, 0, :] += jnp.sum(xn, axis=0)

    in_specs = [pl.BlockSpec((_ROWB, dout), lambda b: (b, 0)),
                pl.BlockSpec((1, 2, dout), lambda b: (b // _BPG, 0, 0)),
                _full((1, dout)), _full((1, dout))]
    out_specs = [pl.BlockSpec((_ROWB, dout // nout), lambda b: (b, 0))
                 for _ in range(nout)]
    out_shape = [jax.ShapeDtypeStruct((2 * _N, dout // nout), jnp.float32)
                 for _ in range(nout)]
    if colsum:
        out_specs.append(pl.BlockSpec((1, 1, dout),
                                      lambda b: (b // _BPG, 0, 0)))
        out_shape.append(jax.ShapeDtypeStruct((2, 1, dout), jnp.float32))
    return pl.pallas_call(body, grid=(_NBLK,), in_specs=in_specs,
                          out_specs=out_specs, out_shape=out_shape,
                          interpret=interpret)


def _make_match(interpret=False):
    def body(cs_ref, m1_ref, m2_ref, out_ref):
        cs = cs_ref[...] / (1.0 * _N)
        r0 = jnp.tanh(jnp.dot(cs[1:2], m2_ref[...],
                              preferred_element_type=jnp.float32))
        r1 = jnp.tanh(jnp.dot(cs[0:1], m1_ref[...],
                              preferred_element_type=jnp.float32))
        out_ref[...] = jnp.concatenate([r0, r1], axis=0)

    return pl.pallas_call(
        body,
        in_specs=[_full((2, 64)), _full((64, 64)), _full((64, 64))],
        out_specs=_full((2, 64)),
        out_shape=jax.ShapeDtypeStruct((2, 64), jnp.float32),
        interpret=interpret)


def _make_abs_colsum(interpret=False):
    def body(a_ref, match_ref, out_ref):
        b = pl.program_id(0)
        x = jnp.abs(a_ref[...] - match_ref[0])

        @pl.when(b % _BPG == 0)
        def _():
            out_ref[...] = jnp.zeros_like(out_ref)
        out_ref[0, 0, :] += jnp.sum(x, axis=0)

    return pl.pallas_call(
        body, grid=(_NBLK,),
        in_specs=[pl.BlockSpec((_ROWB, 64), lambda b: (b, 0)),
                  pl.BlockSpec((1, 1, 64), lambda b: (b // _BPG, 0, 0))],
        out_specs=pl.BlockSpec((1, 1, 64), lambda b: (b // _BPG, 0, 0)),
        out_shape=jax.ShapeDtypeStruct((2, 1, 64), jnp.float32),
        interpret=interpret)


def _make_attention(interpret=False):
    def body(a_ref, match_ref, cs_ref, attw_ref, out_ref):
        b = pl.program_id(0)
        csg = cs_ref[0] / (1.0 * _N)
        t = jnp.tanh(jnp.dot(csg, attw_ref[...],
                             preferred_element_type=jnp.float32))
        x = jnp.abs(a_ref[...] - match_ref[0])
        s = jax.nn.sigmoid(jnp.sum(x * t, axis=1, keepdims=True))
        part = jnp.sum(x * s, axis=0, keepdims=True)

        @pl.when(b % _BPG == 0)
        def _():
            out_ref[...] = jnp.zeros_like(out_ref)
        out_ref[0, :, :] += part

    return pl.pallas_call(
        body, grid=(_NBLK,),
        in_specs=[pl.BlockSpec((_ROWB, 64), lambda b: (b, 0)),
                  pl.BlockSpec((1, 1, 64), lambda b: (b // _BPG, 0, 0)),
                  pl.BlockSpec((1, 1, 64), lambda b: (b // _BPG, 0, 0)),
                  _full((64, 64))],
        out_specs=pl.BlockSpec((1, 1, 64), lambda b: (b // _BPG, 0, 0)),
        out_shape=jax.ShapeDtypeStruct((2, 1, 64), jnp.float32),
        interpret=interpret)


def _make_head(interpret=False):
    def body(p_ref, tnw_ref, tnwbT_ref, tnbT_ref, fcw_ref, fcb_ref,
             scw_ref, scb_ref, avg_ref, score_ref, ged_ref):
        p = p_ref[...]
        p1 = p[0:1]
        p2 = p[1:2]
        tmp = jnp.dot(p1, tnw_ref[...],
                      preferred_element_type=jnp.float32)
        m16 = lax.broadcasted_iota(jnp.int32, (64, 1024), 1)
        i64 = lax.broadcasted_iota(jnp.int32, (64, 1024), 0)
        rep = jnp.where(m16 // 16 == i64, 1.0, 0.0)
        msel = lax.broadcasted_iota(jnp.int32, (1024, 16), 0)
        ksel = lax.broadcasted_iota(jnp.int32, (1024, 16), 1)
        sel = jnp.where(msel % 16 == ksel, 1.0, 0.0)
        p2rep = jnp.dot(p2, rep, preferred_element_type=jnp.float32)
        scoring = jnp.dot(tmp * p2rep, sel,
                          preferred_element_type=jnp.float32)
        cat = jnp.concatenate([p1, p2], axis=1)
        blk = jnp.dot(cat, tnwbT_ref[...],
                      preferred_element_type=jnp.float32)
        s = jnp.maximum(scoring + blk + tnbT_ref[...], 0.0)
        s = jnp.maximum(jnp.dot(s, fcw_ref[...],
                                preferred_element_type=jnp.float32)
                        + fcb_ref[...], 0.0)
        sc = jax.nn.sigmoid(jnp.dot(s, scw_ref[...],
                                    preferred_element_type=jnp.float32)
                            + scb_ref[...])
        score_ref[...] = sc
        ged_ref[...] = -jnp.log(sc) * avg_ref[0, 0]

    return pl.pallas_call(
        body,
        in_specs=[_full((2, 64)), _full((64, 1024)), _full((128, 16)),
                  _full((1, 16)), _full((16, 16)), _full((1, 16)),
                  _full((16, 1)), _full((1, 1)), _full((1, 1))],
        out_specs=[_full((1, 1)), _full((1, 1))],
        out_shape=[jax.ShapeDtypeStruct((1, 1), jnp.float32),
                   jax.ShapeDtypeStruct((1, 1), jnp.float32)],
        interpret=interpret)


_gin_mm = [_make_gin_mm(1, 128, 256), _make_gin_mm(2, 256, 128),
           _make_gin_mm(1, 128, 64)]
_norms = [_make_norm(256, True, 2, False), _make_norm(128, True, 1, False),
          _make_norm(64, False, 1, True)]
_match_k = _make_match()
_abs_colsum_k = _make_abs_colsum()
_attention_k = _make_attention()
_head_k = _make_head()


def kernel(edge_index_1, edge_index_2, features_1, features_2, avg_v,
           g1_W1, g1_b1, g1_W2, g1_b2, g1_gamma, g1_beta, g1_eps,
           g2_W1, g2_b1, g2_W2, g2_b2, g2_gamma, g2_beta, g2_eps,
           g3_W1, g3_b1, g3_W2, g3_b2, g3_gamma, g3_beta, g3_eps,
           m1_W, m2_W, att_W, tn_W, tn_Wb, tn_bias, fc_W, fc_b, sc_W, sc_b):
    f32 = jnp.float32
    ept = _E // _NS

    def padg(v, fill):
        v = v.reshape(_NS, ept)
        return jnp.pad(v, ((0, 0), (0, _TEDGE - ept)),
                       constant_values=fill).reshape(-1)

    srcs = jnp.concatenate([padg(edge_index_1[0], 0),
                            padg(edge_index_2[0] + _N, _N)])
    dsts = jnp.concatenate([padg(edge_index_1[1], _N),
                            padg(edge_index_2[1], _N)])
    row = lambda v: v.reshape(1, -1).astype(f32)
    sca = lambda v: v.reshape(1, 1).astype(f32)

    layer_p = [
        (g1_eps, g1_W1, g1_b1, g1_W2, g1_b2, g1_gamma, g1_beta),
        (g2_eps, g2_W1, g2_b1, g2_W2, g2_b2, g2_gamma, g2_beta),
        (g3_eps, g3_W1, g3_b1, g3_W2, g3_b2, g3_gamma, g3_beta),
    ]

    xh = [jnp.concatenate([features_1, features_2], axis=0)]
    colsum_a = None
    for li in (0, 1, 2):
        eps, W1, b1, W2, b2, gamma, beta = layer_p[li]
        aggs = _segsum(len(xh))(srcs, dsts, *xh)
        if len(xh) == 1:
            aggs = (aggs,) if not isinstance(aggs, (list, tuple)) else aggs
        t, mom = _gin_mm[li](sca(eps), *xh, *aggs, W1, row(b1), W2, row(b2))
        outs = _norms[li](t, mom, row(gamma), row(beta))
        if li == 2:
            a, colsum_a = outs
            xh = [a]
        else:
            xh = list(outs)

    a = xh[0]
    match = _match_k(colsum_a.reshape(2, 64), m1_W, m2_W).reshape(2, 1, 64)
    colsum_x = _abs_colsum_k(a, match)
    p = _attention_k(a, match, colsum_x, att_W)
    score2, ged2 = _head_k(p.reshape(2, 64), tn_W.reshape(64, 64 * 16),
                           tn_Wb.T, tn_bias.reshape(1, 16),
                           fc_W, row(fc_b), sc_W, sca(sc_b), sca(avg_v))
    return score2.reshape(-1), ged2.reshape(-1)


# speedup vs baseline: 1.6237x; 1.6237x over previous
import functools

import jax
import jax.numpy as jnp
from jax import lax
from jax.experimental import pallas as pl
from jax.experimental.pallas import tpu as pltpu
from jax.experimental.pallas import tpu_sc as plsc

_N = 10000
_E = 320000
_NS = 16
_CHUNK = 128
_NCHUNK = _E // _CHUNK
_MAXITER = -(-_NCHUNK // _NS)
_STRIPE = 624
_ZR = 64
_ROWB = 1000
_NBLK = 2 * _N // _ROWB
_BPG = _NBLK // 2


def _make_segsum(nh, interpret=False):
    mesh = plsc.VectorSubcoreMesh(core_axis_name="c", subcore_axis_name="s",
                                  num_cores=2, num_subcores=_NS)
    out_type = [jax.ShapeDtypeStruct((2 * _N, 128), jnp.float32)
                for _ in range(nh)]
    scratch = [
        pltpu.VMEM_SHARED((_N, 128), jnp.float32),
        pltpu.VMEM((_ZR, 128), jnp.float32),
        pltpu.VMEM((_CHUNK,), jnp.int32),
        pltpu.VMEM((_CHUNK,), jnp.int32),
        pltpu.VMEM((_CHUNK,), jnp.int32),
        pltpu.VMEM((_CHUNK,), jnp.int32),
        pltpu.VMEM((_CHUNK, 128), jnp.float32),
        pltpu.SemaphoreType.DMA,
        pltpu.SemaphoreType.DMA,
    ]

    @functools.partial(pl.kernel, mesh=mesh, out_type=out_type,
                       scratch_types=scratch, interpret=interpret)
    def segsum(packed, *rest):
        xtabs = rest[:nh]
        outs = rest[nh:2 * nh]
        agg, zrow, sd0, sd1, src_v, dst_v, rows_v, gsem, isem = rest[2 * nh:]
        c = lax.axis_index("c")
        s = lax.axis_index("s")

        def _zz(i, carry):
            zrow[i // 8, pl.ds((i % 8) * 16, 16)] = jnp.zeros((16,),
                                                              jnp.float32)
            return carry
        lax.fori_loop(0, _ZR * 8, _zz, 0)

        for h in range(nh):
            xt = xtabs[h]
            for q in range(640 // _ZR):
                pltpu.sync_copy(zrow,
                                agg.at[pl.ds(s * _STRIPE + q * _ZR, _ZR)])
            plsc.subcore_barrier()

            def _work(sd):
                for g in range(_CHUNK // 16):
                    v = sd[pl.ds(g * 16, 16)]
                    src_v[pl.ds(g * 16, 16)] = lax.shift_right_logical(v, 14)
                    dst_v[pl.ds(g * 16, 16)] = lax.bitwise_and(v, 16383)
                pltpu.async_copy(xt.at[src_v], rows_v, gsem).wait()
                pltpu.sync_copy(rows_v, agg.at[dst_v], add=True)

            def _step(i, sd_cur, sd_nxt, last):
                k_next = (i + 1) * _NS + s

                @pl.when(k_next < _NCHUNK)
                def _():
                    pltpu.async_copy(
                        packed.at[pl.ds(c * _E + k_next * _CHUNK, _CHUNK)],
                        sd_nxt, isem)
                if last:
                    @pl.when(i * _NS + s < _NCHUNK)
                    def _():
                        _work(sd_cur)
                else:
                    _work(sd_cur)

                @pl.when(k_next < _NCHUNK)
                def _():
                    pltpu.make_async_copy(packed.at[pl.ds(0, _CHUNK)],
                                          sd_nxt, isem).wait()

            pltpu.sync_copy(packed.at[pl.ds(c * _E + s * _CHUNK, _CHUNK)],
                            sd0)

            def _pairs(j, carry):
                _step(2 * j, sd0, sd1, False)
                _step(2 * j + 1, sd1, sd0, False)
                return carry
            lax.fori_loop(0, (_MAXITER - 1) // 2, _pairs, 0)
            _step(_MAXITER - 1, sd0, sd1, True)
            plsc.subcore_barrier()
            pltpu.sync_copy(agg.at[pl.ds(s * _STRIPE, _STRIPE)],
                            outs[h].at[pl.ds(c * _N + s * _STRIPE, _STRIPE)])

            @pl.when(s == _NS - 1)
            def _():
                pltpu.sync_copy(
                    agg.at[pl.ds(_NS * _STRIPE, _N - _NS * _STRIPE)],
                    outs[h].at[pl.ds(c * _N + _NS * _STRIPE,
                                     _N - _NS * _STRIPE)])

    return segsum


_segsum_cache = {}


def _segsum(nh):
    if nh not in _segsum_cache:
        _segsum_cache[nh] = _make_segsum(nh)
    return _segsum_cache[nh]


def _full(shape):
    return pl.BlockSpec(shape, lambda *a: tuple(0 for _ in shape))


def _make_gin_mm(nh, din, dout, interpret=False):
    def body(*refs):
        eps_ref = refs[0]
        x_refs = refs[1:1 + nh]
        agg_refs = refs[1 + nh:1 + 2 * nh]
        w1, b1, w2, b2 = refs[1 + 2 * nh:5 + 2 * nh]
        t_ref, mom = refs[5 + 2 * nh:7 + 2 * nh]
        b = pl.program_id(0)

        eps1 = 1.0 + eps_ref[0, 0]
        parts = [eps1 * x_refs[i][...] + agg_refs[i][...] for i in range(nh)]
        h = parts[0] if nh == 1 else jnp.concatenate(parts, axis=1)
        h1 = jnp.maximum(
            jnp.dot(h, w1[...], preferred_element_type=jnp.float32) + b1[...],
            0.0)
        t = jnp.dot(h1, w2[...], preferred_element_type=jnp.float32) + b2[...]
        t_ref[...] = t

        @pl.when(b % _BPG == 0)
        def _():
            mom[...] = jnp.zeros_like(mom)
        mom[0, 0, :] += jnp.sum(t, axis=0)
        mom[0, 1, :] += jnp.sum(t * t, axis=0)

    hw = 128 if nh > 1 else din
    in_specs = [_full((1, 1))]
    in_specs += [pl.BlockSpec((_ROWB, hw), lambda b: (b, 0))
                 for _ in range(2 * nh)]
    in_specs += [_full((din, dout)), _full((1, dout)),
                 _full((dout, dout)), _full((1, dout))]
    out_specs = [pl.BlockSpec((_ROWB, dout), lambda b: (b, 0)),
                 pl.BlockSpec((1, 2, dout), lambda b: (b // _BPG, 0, 0))]
    out_shape = [jax.ShapeDtypeStruct((2 * _N, dout), jnp.float32),
                 jax.ShapeDtypeStruct((2, 2, dout), jnp.float32)]
    return pl.pallas_call(body, grid=(_NBLK,), in_specs=in_specs,
                          out_specs=out_specs, out_shape=out_shape,
                          interpret=interpret)


def _make_norm(dout, relu, nout, colsum, interpret=False):
    def body(t_ref, mom_ref, g_ref, b_ref, *o_refs):
        b = pl.program_id(0)
        mom = mom_ref[...]
        mu = mom[0, 0:1, :] / (1.0 * _N)
        sq = mom[0, 1:2, :] / (1.0 * _N)
        var = sq - mu * mu
        xn = (t_ref[...] - mu) * lax.rsqrt(var + 1e-5) * g_ref[...] + b_ref[...]
        if relu:
            xn = jnp.maximum(xn, 0.0)
        w = dout // nout
        for i in range(nout):
            o_refs[i][...] = xn[:, i * w:(i + 1) * w]
        if colsum:
            cs = o_refs[nout]

            @pl.when(b % _BPG == 0)
            def _():
                cs[...] = jnp.zeros_like(cs)
            cs[0, 0, :] += jnp.sum(xn, axis=0)

    in_specs = [pl.BlockSpec((_ROWB, dout), lambda b: (b, 0)),
                pl.BlockSpec((1, 2, dout), lambda b: (b // _BPG, 0, 0)),
                _full((1, dout)), _full((1, dout))]
    out_specs = [pl.BlockSpec((_ROWB, dout // nout), lambda b: (b, 0))
                 for _ in range(nout)]
    out_shape = [jax.ShapeDtypeStruct((2 * _N, dout // nout), jnp.float32)
                 for _ in range(nout)]
    if colsum:
        out_specs.append(pl.BlockSpec((1, 1, dout),
                                      lambda b: (b // _BPG, 0, 0)))
        out_shape.append(jax.ShapeDtypeStruct((2, 1, dout), jnp.float32))
    return pl.pallas_call(body, grid=(_NBLK,), in_specs=in_specs,
                          out_specs=out_specs, out_shape=out_shape,
                          interpret=interpret)


def _make_match(interpret=False):
    def body(cs_ref, m1_ref, m2_ref, out_ref):
        cs = cs_ref[...] / (1.0 * _N)
        r0 = jnp.tanh(jnp.dot(cs[1:2], m2_ref[...],
                              preferred_element_type=jnp.float32))
        r1 = jnp.tanh(jnp.dot(cs[0:1], m1_ref[...],
                              preferred_element_type=jnp.float32))
        out_ref[...] = jnp.concatenate([r0, r1], axis=0)

    return pl.pallas_call(
        body,
        in_specs=[_full((2, 64)), _full((64, 64)), _full((64, 64))],
        out_specs=_full((2, 64)),
        out_shape=jax.ShapeDtypeStruct((2, 64), jnp.float32),
        interpret=interpret)


def _make_abs_colsum(interpret=False):
    def body(a_ref, match_ref, out_ref):
        b = pl.program_id(0)
        x = jnp.abs(a_ref[...] - match_ref[0])

        @pl.when(b % _BPG == 0)
        def _():
            out_ref[...] = jnp.zeros_like(out_ref)
        out_ref[0, 0, :] += jnp.sum(x, axis=0)

    return pl.pallas_call(
        body, grid=(_NBLK,),
        in_specs=[pl.BlockSpec((_ROWB, 64), lambda b: (b, 0)),
                  pl.BlockSpec((1, 1, 64), lambda b: (b // _BPG, 0, 0))],
        out_specs=pl.BlockSpec((1, 1, 64), lambda b: (b // _BPG, 0, 0)),
        out_shape=jax.ShapeDtypeStruct((2, 1, 64), jnp.float32),
        interpret=interpret)


def _make_attention(interpret=False):
    def body(a_ref, match_ref, cs_ref, attw_ref, out_ref):
        b = pl.program_id(0)
        csg = cs_ref[0] / (1.0 * _N)
        t = jnp.tanh(jnp.dot(csg, attw_ref[...],
                             preferred_element_type=jnp.float32))
        x = jnp.abs(a_ref[...] - match_ref[0])
        s = jax.nn.sigmoid(jnp.sum(x * t, axis=1, keepdims=True))
        part = jnp.sum(x * s, axis=0, keepdims=True)

        @pl.when(b % _BPG == 0)
        def _():
            out_ref[...] = jnp.zeros_like(out_ref)
        out_ref[0, :, :] += part

    return pl.pallas_call(
        body, grid=(_NBLK,),
        in_specs=[pl.BlockSpec((_ROWB, 64), lambda b: (b, 0)),
                  pl.BlockSpec((1, 1, 64), lambda b: (b // _BPG, 0, 0)),
                  pl.BlockSpec((1, 1, 64), lambda b: (b // _BPG, 0, 0)),
                  _full((64, 64))],
        out_specs=pl.BlockSpec((1, 1, 64), lambda b: (b // _BPG, 0, 0)),
        out_shape=jax.ShapeDtypeStruct((2, 1, 64), jnp.float32),
        interpret=interpret)


def _make_head(interpret=False):
    def body(p_ref, tnw_ref, tnwbT_ref, tnbT_ref, fcw_ref, fcb_ref,
             scw_ref, scb_ref, avg_ref, score_ref, ged_ref):
        p = p_ref[...]
        p1 = p[0:1]
        p2 = p[1:2]
        tmp = jnp.dot(p1, tnw_ref[...],
                      preferred_element_type=jnp.float32)
        m16 = lax.broadcasted_iota(jnp.int32, (64, 1024), 1)
        i64 = lax.broadcasted_iota(jnp.int32, (64, 1024), 0)
        rep = jnp.where(m16 // 16 == i64, 1.0, 0.0)
        msel = lax.broadcasted_iota(jnp.int32, (1024, 16), 0)
        ksel = lax.broadcasted_iota(jnp.int32, (1024, 16), 1)
        sel = jnp.where(msel % 16 == ksel, 1.0, 0.0)
        p2rep = jnp.dot(p2, rep, preferred_element_type=jnp.float32)
        scoring = jnp.dot(tmp * p2rep, sel,
                          preferred_element_type=jnp.float32)
        cat = jnp.concatenate([p1, p2], axis=1)
        blk = jnp.dot(cat, tnwbT_ref[...],
                      preferred_element_type=jnp.float32)
        s = jnp.maximum(scoring + blk + tnbT_ref[...], 0.0)
        s = jnp.maximum(jnp.dot(s, fcw_ref[...],
                                preferred_element_type=jnp.float32)
                        + fcb_ref[...], 0.0)
        sc = jax.nn.sigmoid(jnp.dot(s, scw_ref[...],
                                    preferred_element_type=jnp.float32)
                            + scb_ref[...])
        score_ref[...] = sc
        ged_ref[...] = -jnp.log(sc) * avg_ref[0, 0]

    return pl.pallas_call(
        body,
        in_specs=[_full((2, 64)), _full((64, 1024)), _full((128, 16)),
                  _full((1, 16)), _full((16, 16)), _full((1, 16)),
                  _full((16, 1)), _full((1, 1)), _full((1, 1))],
        out_specs=[_full((1, 1)), _full((1, 1))],
        out_shape=[jax.ShapeDtypeStruct((1, 1), jnp.float32),
                   jax.ShapeDtypeStruct((1, 1), jnp.float32)],
        interpret=interpret)


_gin_mm = [_make_gin_mm(1, 128, 256), _make_gin_mm(2, 256, 128),
           _make_gin_mm(1, 128, 64)]
_norms = [_make_norm(256, True, 2, False), _make_norm(128, True, 1, False),
          _make_norm(64, False, 1, True)]
_match_k = _make_match()
_abs_colsum_k = _make_abs_colsum()
_attention_k = _make_attention()
_head_k = _make_head()


def kernel(edge_index_1, edge_index_2, features_1, features_2, avg_v,
           g1_W1, g1_b1, g1_W2, g1_b2, g1_gamma, g1_beta, g1_eps,
           g2_W1, g2_b1, g2_W2, g2_b2, g2_gamma, g2_beta, g2_eps,
           g3_W1, g3_b1, g3_W2, g3_b2, g3_gamma, g3_beta, g3_eps,
           m1_W, m2_W, att_W, tn_W, tn_Wb, tn_bias, fc_W, fc_b, sc_W, sc_b):
    f32 = jnp.float32
    packed = jnp.concatenate([
        (edge_index_1[0] << 14) | edge_index_1[1],
        ((edge_index_2[0] + _N) << 14) | edge_index_2[1]])
    row = lambda v: v.reshape(1, -1).astype(f32)
    sca = lambda v: v.reshape(1, 1).astype(f32)

    layer_p = [
        (g1_eps, g1_W1, g1_b1, g1_W2, g1_b2, g1_gamma, g1_beta),
        (g2_eps, g2_W1, g2_b1, g2_W2, g2_b2, g2_gamma, g2_beta),
        (g3_eps, g3_W1, g3_b1, g3_W2, g3_b2, g3_gamma, g3_beta),
    ]

    xh = [jnp.concatenate([features_1, features_2], axis=0)]
    colsum_a = None
    for li in (0, 1, 2):
        eps, W1, b1, W2, b2, gamma, beta = layer_p[li]
        aggs = _segsum(len(xh))(packed, *xh)
        if len(xh) == 1:
            aggs = (aggs,) if not isinstance(aggs, (list, tuple)) else aggs
        t, mom = _gin_mm[li](sca(eps), *xh, *aggs, W1, row(b1), W2, row(b2))
        outs = _norms[li](t, mom, row(gamma), row(beta))
        if li == 2:
            a, colsum_a = outs
            xh = [a]
        else:
            xh = list(outs)

    a = xh[0]
    match = _match_k(colsum_a.reshape(2, 64), m1_W, m2_W).reshape(2, 1, 64)
    colsum_x = _abs_colsum_k(a, match)
    p = _attention_k(a, match, colsum_x, att_W)
    score2, ged2 = _head_k(p.reshape(2, 64), tn_W.reshape(64, 64 * 16),
                           tn_Wb.T, tn_bias.reshape(1, 16),
                           fc_W, row(fc_b), sc_W, sca(sc_b), sca(avg_v))
    return score2.reshape(-1), ged2.reshape(-1)


# speedup vs baseline: 2.1527x; 1.3258x over previous
import functools

import jax
import jax.numpy as jnp
from jax import lax
from jax.experimental import pallas as pl
from jax.experimental.pallas import tpu as pltpu
from jax.experimental.pallas import tpu_sc as plsc

_N = 10000
_E = 320000
_NS = 16
_CHUNK = 128
_NCHUNK = _E // _CHUNK
_MAXITER = -(-_NCHUNK // _NS)
_MAXITER_TAIL = (_MAXITER - 1) * _NS
_STRIPE = 624
_ZR = 64
_ROWB = 1000
_NBLK = 2 * _N // _ROWB
_BPG = _NBLK // 2


def _make_segsum(nh, interpret=False):
    mesh = plsc.VectorSubcoreMesh(core_axis_name="c", subcore_axis_name="s",
                                  num_cores=2, num_subcores=_NS)
    out_type = [jax.ShapeDtypeStruct((2 * _N, 128), jnp.float32)
                for _ in range(nh)]
    scratch = [
        pltpu.VMEM_SHARED((_N, 128), jnp.float32),
        pltpu.VMEM((_ZR, 128), jnp.float32),
        pltpu.VMEM((_CHUNK,), jnp.int32),
        pltpu.VMEM((_CHUNK,), jnp.int32),
        pltpu.VMEM((_CHUNK,), jnp.int32),
        pltpu.VMEM((_CHUNK,), jnp.int32),
        pltpu.VMEM((_CHUNK,), jnp.int32),
        pltpu.VMEM((_CHUNK,), jnp.int32),
        pltpu.VMEM((_CHUNK, 128), jnp.float32),
        pltpu.VMEM((_CHUNK, 128), jnp.float32),
        pltpu.SemaphoreType.DMA,
        pltpu.SemaphoreType.DMA,
        pltpu.SemaphoreType.DMA,
    ]

    @functools.partial(pl.kernel, mesh=mesh, out_type=out_type,
                       scratch_types=scratch, interpret=interpret)
    def segsum(packed, *rest):
        xtabs = rest[:nh]
        outs = rest[nh:2 * nh]
        (agg, zrow, sd0, sd1, srcA, dstA, srcB, dstB, rowsA, rowsB,
         gsemA, gsemB, isem) = rest[2 * nh:]
        c = lax.axis_index("c")
        s = lax.axis_index("s")

        def _zz(i, carry):
            zrow[i // 8, pl.ds((i % 8) * 16, 16)] = jnp.zeros((16,),
                                                              jnp.float32)
            return carry
        lax.fori_loop(0, _ZR * 8, _zz, 0)

        for h in range(nh):
            xt = xtabs[h]
            for q in range(640 // _ZR):
                pltpu.sync_copy(zrow,
                                agg.at[pl.ds(s * _STRIPE + q * _ZR, _ZR)])
            plsc.subcore_barrier()

            def _unpack(sd, src_v, dst_v):
                for g in range(_CHUNK // 16):
                    v = sd[pl.ds(g * 16, 16)]
                    src_v[pl.ds(g * 16, 16)] = lax.shift_right_logical(v, 14)
                    dst_v[pl.ds(g * 16, 16)] = lax.bitwise_and(v, 16383)

            def _step(i, sd_nxt, src_c, dst_c, rows_c, gsem_c,
                      src_n, dst_n, rows_n, gsem_n):
                k_next = (i + 1) * _NS + s

                @pl.when(k_next < _NCHUNK)
                def _():
                    pltpu.async_copy(
                        packed.at[pl.ds(c * _E + k_next * _CHUNK, _CHUNK)],
                        sd_nxt, isem)
                    pltpu.make_async_copy(packed.at[pl.ds(0, _CHUNK)],
                                          sd_nxt, isem).wait()
                    _unpack(sd_nxt, src_n, dst_n)
                pltpu.make_async_copy(xt.at[pl.ds(0, _CHUNK)], rows_c,
                                      gsem_c).wait()

                @pl.when(k_next < _NCHUNK)
                def _():
                    pltpu.async_copy(xt.at[src_n], rows_n, gsem_n)
                pltpu.sync_copy(rows_c, agg.at[dst_c], add=True)

            pltpu.sync_copy(packed.at[pl.ds(c * _E + s * _CHUNK, _CHUNK)],
                            sd0)
            _unpack(sd0, srcA, dstA)
            pltpu.async_copy(xt.at[srcA], rowsA, gsemA)

            def _pairs(j, carry):
                _step(2 * j, sd1, srcA, dstA, rowsA, gsemA,
                      srcB, dstB, rowsB, gsemB)
                _step(2 * j + 1, sd0, srcB, dstB, rowsB, gsemB,
                      srcA, dstA, rowsA, gsemA)
                return carry
            lax.fori_loop(0, (_MAXITER - 1) // 2, _pairs, 0)

            @pl.when(_MAXITER_TAIL + s < _NCHUNK)
            def _():
                pltpu.make_async_copy(xt.at[pl.ds(0, _CHUNK)], rowsA,
                                      gsemA).wait()
                pltpu.sync_copy(rowsA, agg.at[dstA], add=True)
            plsc.subcore_barrier()
            pltpu.sync_copy(agg.at[pl.ds(s * _STRIPE, _STRIPE)],
                            outs[h].at[pl.ds(c * _N + s * _STRIPE, _STRIPE)])

            @pl.when(s == _NS - 1)
            def _():
                pltpu.sync_copy(
                    agg.at[pl.ds(_NS * _STRIPE, _N - _NS * _STRIPE)],
                    outs[h].at[pl.ds(c * _N + _NS * _STRIPE,
                                     _N - _NS * _STRIPE)])

    return segsum


_segsum_cache = {}


def _segsum(nh):
    if nh not in _segsum_cache:
        _segsum_cache[nh] = _make_segsum(nh)
    return _segsum_cache[nh]


def _full(shape):
    return pl.BlockSpec(shape, lambda *a: tuple(0 for _ in shape))


def _make_gin_mm(nh, din, dout, interpret=False):
    def body(*refs):
        eps_ref = refs[0]
        x_refs = refs[1:1 + nh]
        agg_refs = refs[1 + nh:1 + 2 * nh]
        w1, b1, w2, b2 = refs[1 + 2 * nh:5 + 2 * nh]
        t_ref, mom = refs[5 + 2 * nh:7 + 2 * nh]
        b = pl.program_id(0)

        eps1 = 1.0 + eps_ref[0, 0]
        parts = [eps1 * x_refs[i][...] + agg_refs[i][...] for i in range(nh)]
        h = parts[0] if nh == 1 else jnp.concatenate(parts, axis=1)
        h1 = jnp.maximum(
            jnp.dot(h, w1[...], preferred_element_type=jnp.float32) + b1[...],
            0.0)
        t = jnp.dot(h1, w2[...], preferred_element_type=jnp.float32) + b2[...]
        t_ref[...] = t

        @pl.when(b % _BPG == 0)
        def _():
            mom[...] = jnp.zeros_like(mom)
        mom[0, 0, :] += jnp.sum(t, axis=0)
        mom[0, 1, :] += jnp.sum(t * t, axis=0)

    hw = 128 if nh > 1 else din
    in_specs = [_full((1, 1))]
    in_specs += [pl.BlockSpec((_ROWB, hw), lambda b: (b, 0))
                 for _ in range(2 * nh)]
    in_specs += [_full((din, dout)), _full((1, dout)),
                 _full((dout, dout)), _full((1, dout))]
    out_specs = [pl.BlockSpec((_ROWB, dout), lambda b: (b, 0)),
                 pl.BlockSpec((1, 2, dout), lambda b: (b // _BPG, 0, 0))]
    out_shape = [jax.ShapeDtypeStruct((2 * _N, dout), jnp.float32),
                 jax.ShapeDtypeStruct((2, 2, dout), jnp.float32)]
    return pl.pallas_call(body, grid=(_NBLK,), in_specs=in_specs,
                          out_specs=out_specs, out_shape=out_shape,
                          interpret=interpret)


def _make_norm(dout, relu, nout, colsum, interpret=False):
    def body(t_ref, mom_ref, g_ref, b_ref, *o_refs):
        b = pl.program_id(0)
        mom = mom_ref[...]
        mu = mom[0, 0:1, :] / (1.0 * _N)
        sq = mom[0, 1:2, :] / (1.0 * _N)
        var = sq - mu * mu
        xn = (t_ref[...] - mu) * lax.rsqrt(var + 1e-5) * g_ref[...] + b_ref[...]
        if relu:
            xn = jnp.maximum(xn, 0.0)
        w = dout // nout
        for i in range(nout):
            o_refs[i][...] = xn[:, i * w:(i + 1) * w]
        if colsum:
            cs = o_refs[nout]

            @pl.when(b % _BPG == 0)
            def _():
                cs[...] = jnp.zeros_like(cs)
            cs[0, 0, :] += jnp.sum(xn, axis=0)

    in_specs = [pl.BlockSpec((_ROWB, dout), lambda b: (b, 0)),
                pl.BlockSpec((1, 2, dout), lambda b: (b // _BPG, 0, 0)),
                _full((1, dout)), _full((1, dout))]
    out_specs = [pl.BlockSpec((_ROWB, dout // nout), lambda b: (b, 0))
                 for _ in range(nout)]
    out_shape = [jax.ShapeDtypeStruct((2 * _N, dout // nout), jnp.float32)
                 for _ in range(nout)]
    if colsum:
        out_specs.append(pl.BlockSpec((1, 1, dout),
                                      lambda b: (b // _BPG, 0, 0)))
        out_shape.append(jax.ShapeDtypeStruct((2, 1, dout), jnp.float32))
    return pl.pallas_call(body, grid=(_NBLK,), in_specs=in_specs,
                          out_specs=out_specs, out_shape=out_shape,
                          interpret=interpret)


def _make_match(interpret=False):
    def body(cs_ref, m1_ref, m2_ref, out_ref):
        cs = cs_ref[...] / (1.0 * _N)
        r0 = jnp.tanh(jnp.dot(cs[1:2], m2_ref[...],
                              preferred_element_type=jnp.float32))
        r1 = jnp.tanh(jnp.dot(cs[0:1], m1_ref[...],
                              preferred_element_type=jnp.float32))
        out_ref[...] = jnp.concatenate([r0, r1], axis=0)

    return pl.pallas_call(
        body,
        in_specs=[_full((2, 64)), _full((64, 64)), _full((64, 64))],
        out_specs=_full((2, 64)),
        out_shape=jax.ShapeDtypeStruct((2, 64), jnp.float32),
        interpret=interpret)


def _make_abs_colsum(interpret=False):
    def body(a_ref, match_ref, out_ref):
        b = pl.program_id(0)
        x = jnp.abs(a_ref[...] - match_ref[0])

        @pl.when(b % _BPG == 0)
        def _():
            out_ref[...] = jnp.zeros_like(out_ref)
        out_ref[0, 0, :] += jnp.sum(x, axis=0)

    return pl.pallas_call(
        body, grid=(_NBLK,),
        in_specs=[pl.BlockSpec((_ROWB, 64), lambda b: (b, 0)),
                  pl.BlockSpec((1, 1, 64), lambda b: (b // _BPG, 0, 0))],
        out_specs=pl.BlockSpec((1, 1, 64), lambda b: (b // _BPG, 0, 0)),
        out_shape=jax.ShapeDtypeStruct((2, 1, 64), jnp.float32),
        interpret=interpret)


def _make_attention(interpret=False):
    def body(a_ref, match_ref, cs_ref, attw_ref, out_ref):
        b = pl.program_id(0)
        csg = cs_ref[0] / (1.0 * _N)
        t = jnp.tanh(jnp.dot(csg, attw_ref[...],
                             preferred_element_type=jnp.float32))
        x = jnp.abs(a_ref[...] - match_ref[0])
        s = jax.nn.sigmoid(jnp.sum(x * t, axis=1, keepdims=True))
        part = jnp.sum(x * s, axis=0, keepdims=True)

        @pl.when(b % _BPG == 0)
        def _():
            out_ref[...] = jnp.zeros_like(out_ref)
        out_ref[0, :, :] += part

    return pl.pallas_call(
        body, grid=(_NBLK,),
        in_specs=[pl.BlockSpec((_ROWB, 64), lambda b: (b, 0)),
                  pl.BlockSpec((1, 1, 64), lambda b: (b // _BPG, 0, 0)),
                  pl.BlockSpec((1, 1, 64), lambda b: (b // _BPG, 0, 0)),
                  _full((64, 64))],
        out_specs=pl.BlockSpec((1, 1, 64), lambda b: (b // _BPG, 0, 0)),
        out_shape=jax.ShapeDtypeStruct((2, 1, 64), jnp.float32),
        interpret=interpret)


def _make_head(interpret=False):
    def body(p_ref, tnw_ref, tnwbT_ref, tnbT_ref, fcw_ref, fcb_ref,
             scw_ref, scb_ref, avg_ref, score_ref, ged_ref):
        p = p_ref[...]
        p1 = p[0:1]
        p2 = p[1:2]
        tmp = jnp.dot(p1, tnw_ref[...],
                      preferred_element_type=jnp.float32)
        m16 = lax.broadcasted_iota(jnp.int32, (64, 1024), 1)
        i64 = lax.broadcasted_iota(jnp.int32, (64, 1024), 0)
        rep = jnp.where(m16 // 16 == i64, 1.0, 0.0)
        msel = lax.broadcasted_iota(jnp.int32, (1024, 16), 0)
        ksel = lax.broadcasted_iota(jnp.int32, (1024, 16), 1)
        sel = jnp.where(msel % 16 == ksel, 1.0, 0.0)
        p2rep = jnp.dot(p2, rep, preferred_element_type=jnp.float32)
        scoring = jnp.dot(tmp * p2rep, sel,
                          preferred_element_type=jnp.float32)
        cat = jnp.concatenate([p1, p2], axis=1)
        blk = jnp.dot(cat, tnwbT_ref[...],
                      preferred_element_type=jnp.float32)
        s = jnp.maximum(scoring + blk + tnbT_ref[...], 0.0)
        s = jnp.maximum(jnp.dot(s, fcw_ref[...],
                                preferred_element_type=jnp.float32)
                        + fcb_ref[...], 0.0)
        sc = jax.nn.sigmoid(jnp.dot(s, scw_ref[...],
                                    preferred_element_type=jnp.float32)
                            + scb_ref[...])
        score_ref[...] = sc
        ged_ref[...] = -jnp.log(sc) * avg_ref[0, 0]

    return pl.pallas_call(
        body,
        in_specs=[_full((2, 64)), _full((64, 1024)), _full((128, 16)),
                  _full((1, 16)), _full((16, 16)), _full((1, 16)),
                  _full((16, 1)), _full((1, 1)), _full((1, 1))],
        out_specs=[_full((1, 1)), _full((1, 1))],
        out_shape=[jax.ShapeDtypeStruct((1, 1), jnp.float32),
                   jax.ShapeDtypeStruct((1, 1), jnp.float32)],
        interpret=interpret)


_gin_mm = [_make_gin_mm(1, 128, 256), _make_gin_mm(2, 256, 128),
           _make_gin_mm(1, 128, 64)]
_norms = [_make_norm(256, True, 2, False), _make_norm(128, True, 1, False),
          _make_norm(64, False, 1, True)]
_match_k = _make_match()
_abs_colsum_k = _make_abs_colsum()
_attention_k = _make_attention()
_head_k = _make_head()


def kernel(edge_index_1, edge_index_2, features_1, features_2, avg_v,
           g1_W1, g1_b1, g1_W2, g1_b2, g1_gamma, g1_beta, g1_eps,
           g2_W1, g2_b1, g2_W2, g2_b2, g2_gamma, g2_beta, g2_eps,
           g3_W1, g3_b1, g3_W2, g3_b2, g3_gamma, g3_beta, g3_eps,
           m1_W, m2_W, att_W, tn_W, tn_Wb, tn_bias, fc_W, fc_b, sc_W, sc_b):
    f32 = jnp.float32
    packed = jnp.concatenate([
        (edge_index_1[0] << 14) | edge_index_1[1],
        ((edge_index_2[0] + _N) << 14) | edge_index_2[1]])
    row = lambda v: v.reshape(1, -1).astype(f32)
    sca = lambda v: v.reshape(1, 1).astype(f32)

    layer_p = [
        (g1_eps, g1_W1, g1_b1, g1_W2, g1_b2, g1_gamma, g1_beta),
        (g2_eps, g2_W1, g2_b1, g2_W2, g2_b2, g2_gamma, g2_beta),
        (g3_eps, g3_W1, g3_b1, g3_W2, g3_b2, g3_gamma, g3_beta),
    ]

    xh = [jnp.concatenate([features_1, features_2], axis=0)]
    colsum_a = None
    for li in (0, 1, 2):
        eps, W1, b1, W2, b2, gamma, beta = layer_p[li]
        aggs = _segsum(len(xh))(packed, *xh)
        if len(xh) == 1:
            aggs = (aggs,) if not isinstance(aggs, (list, tuple)) else aggs
        t, mom = _gin_mm[li](sca(eps), *xh, *aggs, W1, row(b1), W2, row(b2))
        outs = _norms[li](t, mom, row(gamma), row(beta))
        if li == 2:
            a, colsum_a = outs
            xh = [a]
        else:
            xh = list(outs)

    a = xh[0]
    match = _match_k(colsum_a.reshape(2, 64), m1_W, m2_W).reshape(2, 1, 64)
    colsum_x = _abs_colsum_k(a, match)
    p = _attention_k(a, match, colsum_x, att_W)
    score2, ged2 = _head_k(p.reshape(2, 64), tn_W.reshape(64, 64 * 16),
                           tn_Wb.T, tn_bias.reshape(1, 16),
                           fc_W, row(fc_b), sc_W, sca(sc_b), sca(avg_v))
    return score2.reshape(-1), ged2.reshape(-1)


# speedup vs baseline: 2.1603x; 1.0035x over previous
import functools

import jax
import jax.numpy as jnp
from jax import lax
from jax.experimental import pallas as pl
from jax.experimental.pallas import tpu as pltpu
from jax.experimental.pallas import tpu_sc as plsc

_N = 10000
_E = 320000
_NS = 16
_CHUNK = 128
_NCHUNK = _E // _CHUNK
_MAXITER = -(-_NCHUNK // _NS)
_MAXITER_TAIL = (_MAXITER - 1) * _NS
_STRIPE = 624
_ZR = 64
_ROWB = 1000
_NBLK = 2 * _N // _ROWB
_BPG = _NBLK // 2


def _make_segsum(nh, interpret=False):
    mesh = plsc.VectorSubcoreMesh(core_axis_name="c", subcore_axis_name="s",
                                  num_cores=2, num_subcores=_NS)
    out_type = [jax.ShapeDtypeStruct((2 * _N, 128), jnp.float32)
                for _ in range(nh)]
    scratch = [
        pltpu.VMEM_SHARED((_N, 128), jnp.float32),
        pltpu.VMEM((_ZR, 128), jnp.float32),
        pltpu.VMEM((_CHUNK,), jnp.int32),
        pltpu.VMEM((_CHUNK,), jnp.int32),
        pltpu.VMEM((_CHUNK,), jnp.int32),
        pltpu.VMEM((_CHUNK,), jnp.int32),
        pltpu.VMEM((_CHUNK,), jnp.int32),
        pltpu.VMEM((_CHUNK,), jnp.int32),
        pltpu.VMEM((_CHUNK, 128), jnp.float32),
        pltpu.VMEM((_CHUNK, 128), jnp.float32),
        pltpu.SemaphoreType.DMA,
        pltpu.SemaphoreType.DMA,
        pltpu.SemaphoreType.DMA,
        pltpu.SemaphoreType.DMA,
        pltpu.SemaphoreType.DMA,
    ]

    @functools.partial(pl.kernel, mesh=mesh, out_type=out_type,
                       scratch_types=scratch, interpret=interpret)
    def segsum(packed, *rest):
        xtabs = rest[:nh]
        outs = rest[nh:2 * nh]
        (agg, zrow, sd0, sd1, srcA, dstA, srcB, dstB, rowsA, rowsB,
         gsemA, gsemB, ssemA, ssemB, isem) = rest[2 * nh:]
        c = lax.axis_index("c")
        s = lax.axis_index("s")

        def _zz(i, carry):
            zrow[i // 8, pl.ds((i % 8) * 16, 16)] = jnp.zeros((16,),
                                                              jnp.float32)
            return carry
        lax.fori_loop(0, _ZR * 8, _zz, 0)

        for h in range(nh):
            xt = xtabs[h]
            for q in range(640 // _ZR):
                pltpu.sync_copy(zrow,
                                agg.at[pl.ds(s * _STRIPE + q * _ZR, _ZR)])
            plsc.subcore_barrier()

            def _unpack(sd, src_v, dst_v):
                for g in range(_CHUNK // 16):
                    v = sd[pl.ds(g * 16, 16)]
                    src_v[pl.ds(g * 16, 16)] = lax.shift_right_logical(v, 14)
                    dst_v[pl.ds(g * 16, 16)] = lax.bitwise_and(v, 16383)

            def _prefetch(i, sd_nxt, src_n, dst_n):
                k_next = (i + 1) * _NS + s

                @pl.when(k_next < _NCHUNK)
                def _():
                    pltpu.async_copy(
                        packed.at[pl.ds(c * _E + k_next * _CHUNK, _CHUNK)],
                        sd_nxt, isem)
                    pltpu.make_async_copy(packed.at[pl.ds(0, _CHUNK)],
                                          sd_nxt, isem).wait()
                    _unpack(sd_nxt, src_n, dst_n)

            def _step(i, sd_nxt, src_c, dst_c, rows_c, gsem_c, ssem_c,
                      src_n, dst_n, rows_n, gsem_n, ssem_n):
                k_next = (i + 1) * _NS + s

                @pl.when(k_next < _NCHUNK)
                def _():
                    pltpu.async_copy(
                        packed.at[pl.ds(c * _E + k_next * _CHUNK, _CHUNK)],
                        sd_nxt, isem)
                pltpu.make_async_copy(rows_n, agg.at[pl.ds(0, _CHUNK)],
                                      ssem_n).wait()

                @pl.when(k_next < _NCHUNK)
                def _():
                    pltpu.make_async_copy(packed.at[pl.ds(0, _CHUNK)],
                                          sd_nxt, isem).wait()
                    _unpack(sd_nxt, src_n, dst_n)
                pltpu.make_async_copy(xt.at[pl.ds(0, _CHUNK)], rows_c,
                                      gsem_c).wait()

                @pl.when(k_next < _NCHUNK)
                def _():
                    pltpu.async_copy(xt.at[src_n], rows_n, gsem_n)
                pltpu.async_copy(rows_c, agg.at[dst_c], ssem_c, add=True)

            pltpu.sync_copy(packed.at[pl.ds(c * _E + s * _CHUNK, _CHUNK)],
                            sd0)
            _unpack(sd0, srcA, dstA)
            pltpu.async_copy(xt.at[srcA], rowsA, gsemA)
            _prefetch(0, sd1, srcB, dstB)
            pltpu.make_async_copy(xt.at[pl.ds(0, _CHUNK)], rowsA,
                                  gsemA).wait()
            pltpu.async_copy(xt.at[srcB], rowsB, gsemB)
            pltpu.async_copy(rowsA, agg.at[dstA], ssemA, add=True)

            def _pairs(j, carry):
                _step(2 * j + 1, sd0, srcB, dstB, rowsB, gsemB, ssemB,
                      srcA, dstA, rowsA, gsemA, ssemA)
                _step(2 * j + 2, sd1, srcA, dstA, rowsA, gsemA, ssemA,
                      srcB, dstB, rowsB, gsemB, ssemB)
                return carry
            lax.fori_loop(0, (_MAXITER - 3) // 2, _pairs, 0)
            _step(_MAXITER - 2, sd0, srcB, dstB, rowsB, gsemB, ssemB,
                  srcA, dstA, rowsA, gsemA, ssemA)
            pltpu.make_async_copy(rowsB, agg.at[pl.ds(0, _CHUNK)],
                                  ssemB).wait()

            @pl.when(_MAXITER_TAIL + s < _NCHUNK)
            def _():
                pltpu.make_async_copy(xt.at[pl.ds(0, _CHUNK)], rowsA,
                                      gsemA).wait()
                pltpu.sync_copy(rowsA, agg.at[dstA], add=True)
            plsc.subcore_barrier()
            pltpu.sync_copy(agg.at[pl.ds(s * _STRIPE, _STRIPE)],
                            outs[h].at[pl.ds(c * _N + s * _STRIPE, _STRIPE)])

            @pl.when(s == _NS - 1)
            def _():
                pltpu.sync_copy(
                    agg.at[pl.ds(_NS * _STRIPE, _N - _NS * _STRIPE)],
                    outs[h].at[pl.ds(c * _N + _NS * _STRIPE,
                                     _N - _NS * _STRIPE)])

    return segsum


_segsum_cache = {}


def _segsum(nh):
    if nh not in _segsum_cache:
        _segsum_cache[nh] = _make_segsum(nh)
    return _segsum_cache[nh]


def _full(shape):
    return pl.BlockSpec(shape, lambda *a: tuple(0 for _ in shape))


def _make_gin_mm(nh, din, dout, interpret=False):
    def body(*refs):
        eps_ref = refs[0]
        x_refs = refs[1:1 + nh]
        agg_refs = refs[1 + nh:1 + 2 * nh]
        w1, b1, w2, b2 = refs[1 + 2 * nh:5 + 2 * nh]
        t_ref, mom = refs[5 + 2 * nh:7 + 2 * nh]
        b = pl.program_id(0)

        eps1 = 1.0 + eps_ref[0, 0]
        parts = [eps1 * x_refs[i][...] + agg_refs[i][...] for i in range(nh)]
        h = parts[0] if nh == 1 else jnp.concatenate(parts, axis=1)
        h1 = jnp.maximum(
            jnp.dot(h, w1[...], preferred_element_type=jnp.float32) + b1[...],
            0.0)
        t = jnp.dot(h1, w2[...], preferred_element_type=jnp.float32) + b2[...]
        t_ref[...] = t

        @pl.when(b % _BPG == 0)
        def _():
            mom[...] = jnp.zeros_like(mom)
        mom[0, 0, :] += jnp.sum(t, axis=0)
        mom[0, 1, :] += jnp.sum(t * t, axis=0)

    hw = 128 if nh > 1 else din
    in_specs = [_full((1, 1))]
    in_specs += [pl.BlockSpec((_ROWB, hw), lambda b: (b, 0))
                 for _ in range(2 * nh)]
    in_specs += [_full((din, dout)), _full((1, dout)),
                 _full((dout, dout)), _full((1, dout))]
    out_specs = [pl.BlockSpec((_ROWB, dout), lambda b: (b, 0)),
                 pl.BlockSpec((1, 2, dout), lambda b: (b // _BPG, 0, 0))]
    out_shape = [jax.ShapeDtypeStruct((2 * _N, dout), jnp.float32),
                 jax.ShapeDtypeStruct((2, 2, dout), jnp.float32)]
    return pl.pallas_call(body, grid=(_NBLK,), in_specs=in_specs,
                          out_specs=out_specs, out_shape=out_shape,
                          interpret=interpret)


def _make_norm(dout, relu, nout, colsum, interpret=False):
    def body(t_ref, mom_ref, g_ref, b_ref, *o_refs):
        b = pl.program_id(0)
        mom = mom_ref[...]
        mu = mom[0, 0:1, :] / (1.0 * _N)
        sq = mom[0, 1:2, :] / (1.0 * _N)
        var = sq - mu * mu
        xn = (t_ref[...] - mu) * lax.rsqrt(var + 1e-5) * g_ref[...] + b_ref[...]
        if relu:
            xn = jnp.maximum(xn, 0.0)
        w = dout // nout
        for i in range(nout):
            o_refs[i][...] = xn[:, i * w:(i + 1) * w]
        if colsum:
            cs = o_refs[nout]

            @pl.when(b % _BPG == 0)
            def _():
                cs[...] = jnp.zeros_like(cs)
            cs[0, 0, :] += jnp.sum(xn, axis=0)

    in_specs = [pl.BlockSpec((_ROWB, dout), lambda b: (b, 0)),
                pl.BlockSpec((1, 2, dout), lambda b: (b // _BPG, 0, 0)),
                _full((1, dout)), _full((1, dout))]
    out_specs = [pl.BlockSpec((_ROWB, dout // nout), lambda b: (b, 0))
                 for _ in range(nout)]
    out_shape = [jax.ShapeDtypeStruct((2 * _N, dout // nout), jnp.float32)
                 for _ in range(nout)]
    if colsum:
        out_specs.append(pl.BlockSpec((1, 1, dout),
                                      lambda b: (b // _BPG, 0, 0)))
        out_shape.append(jax.ShapeDtypeStruct((2, 1, dout), jnp.float32))
    return pl.pallas_call(body, grid=(_NBLK,), in_specs=in_specs,
                          out_specs=out_specs, out_shape=out_shape,
                          interpret=interpret)


def _make_match(interpret=False):
    def body(cs_ref, m1_ref, m2_ref, out_ref):
        cs = cs_ref[...] / (1.0 * _N)
        r0 = jnp.tanh(jnp.dot(cs[1:2], m2_ref[...],
                              preferred_element_type=jnp.float32))
        r1 = jnp.tanh(jnp.dot(cs[0:1], m1_ref[...],
                              preferred_element_type=jnp.float32))
        out_ref[...] = jnp.concatenate([r0, r1], axis=0)

    return pl.pallas_call(
        body,
        in_specs=[_full((2, 64)), _full((64, 64)), _full((64, 64))],
        out_specs=_full((2, 64)),
        out_shape=jax.ShapeDtypeStruct((2, 64), jnp.float32),
        interpret=interpret)


def _make_abs_colsum(interpret=False):
    def body(a_ref, match_ref, out_ref):
        b = pl.program_id(0)
        x = jnp.abs(a_ref[...] - match_ref[0])

        @pl.when(b % _BPG == 0)
        def _():
            out_ref[...] = jnp.zeros_like(out_ref)
        out_ref[0, 0, :] += jnp.sum(x, axis=0)

    return pl.pallas_call(
        body, grid=(_NBLK,),
        in_specs=[pl.BlockSpec((_ROWB, 64), lambda b: (b, 0)),
                  pl.BlockSpec((1, 1, 64), lambda b: (b // _BPG, 0, 0))],
        out_specs=pl.BlockSpec((1, 1, 64), lambda b: (b // _BPG, 0, 0)),
        out_shape=jax.ShapeDtypeStruct((2, 1, 64), jnp.float32),
        interpret=interpret)


def _make_attention(interpret=False):
    def body(a_ref, match_ref, cs_ref, attw_ref, out_ref):
        b = pl.program_id(0)
        csg = cs_ref[0] / (1.0 * _N)
        t = jnp.tanh(jnp.dot(csg, attw_ref[...],
                             preferred_element_type=jnp.float32))
        x = jnp.abs(a_ref[...] - match_ref[0])
        s = jax.nn.sigmoid(jnp.sum(x * t, axis=1, keepdims=True))
        part = jnp.sum(x * s, axis=0, keepdims=True)

        @pl.when(b % _BPG == 0)
        def _():
            out_ref[...] = jnp.zeros_like(out_ref)
        out_ref[0, :, :] += part

    return pl.pallas_call(
        body, grid=(_NBLK,),
        in_specs=[pl.BlockSpec((_ROWB, 64), lambda b: (b, 0)),
                  pl.BlockSpec((1, 1, 64), lambda b: (b // _BPG, 0, 0)),
                  pl.BlockSpec((1, 1, 64), lambda b: (b // _BPG, 0, 0)),
                  _full((64, 64))],
        out_specs=pl.BlockSpec((1, 1, 64), lambda b: (b // _BPG, 0, 0)),
        out_shape=jax.ShapeDtypeStruct((2, 1, 64), jnp.float32),
        interpret=interpret)


def _make_head(interpret=False):
    def body(p_ref, tnw_ref, tnwbT_ref, tnbT_ref, fcw_ref, fcb_ref,
             scw_ref, scb_ref, avg_ref, score_ref, ged_ref):
        p = p_ref[...]
        p1 = p[0:1]
        p2 = p[1:2]
        tmp = jnp.dot(p1, tnw_ref[...],
                      preferred_element_type=jnp.float32)
        m16 = lax.broadcasted_iota(jnp.int32, (64, 1024), 1)
        i64 = lax.broadcasted_iota(jnp.int32, (64, 1024), 0)
        rep = jnp.where(m16 // 16 == i64, 1.0, 0.0)
        msel = lax.broadcasted_iota(jnp.int32, (1024, 16), 0)
        ksel = lax.broadcasted_iota(jnp.int32, (1024, 16), 1)
        sel = jnp.where(msel % 16 == ksel, 1.0, 0.0)
        p2rep = jnp.dot(p2, rep, preferred_element_type=jnp.float32)
        scoring = jnp.dot(tmp * p2rep, sel,
                          preferred_element_type=jnp.float32)
        cat = jnp.concatenate([p1, p2], axis=1)
        blk = jnp.dot(cat, tnwbT_ref[...],
                      preferred_element_type=jnp.float32)
        s = jnp.maximum(scoring + blk + tnbT_ref[...], 0.0)
        s = jnp.maximum(jnp.dot(s, fcw_ref[...],
                                preferred_element_type=jnp.float32)
                        + fcb_ref[...], 0.0)
        sc = jax.nn.sigmoid(jnp.dot(s, scw_ref[...],
                                    preferred_element_type=jnp.float32)
                            + scb_ref[...])
        score_ref[...] = sc
        ged_ref[...] = -jnp.log(sc) * avg_ref[0, 0]

    return pl.pallas_call(
        body,
        in_specs=[_full((2, 64)), _full((64, 1024)), _full((128, 16)),
                  _full((1, 16)), _full((16, 16)), _full((1, 16)),
                  _full((16, 1)), _full((1, 1)), _full((1, 1))],
        out_specs=[_full((1, 1)), _full((1, 1))],
        out_shape=[jax.ShapeDtypeStruct((1, 1), jnp.float32),
                   jax.ShapeDtypeStruct((1, 1), jnp.float32)],
        interpret=interpret)


_gin_mm = [_make_gin_mm(1, 128, 256), _make_gin_mm(2, 256, 128),
           _make_gin_mm(1, 128, 64)]
_norms = [_make_norm(256, True, 2, False), _make_norm(128, True, 1, False),
          _make_norm(64, False, 1, True)]
_match_k = _make_match()
_abs_colsum_k = _make_abs_colsum()
_attention_k = _make_attention()
_head_k = _make_head()


def kernel(edge_index_1, edge_index_2, features_1, features_2, avg_v,
           g1_W1, g1_b1, g1_W2, g1_b2, g1_gamma, g1_beta, g1_eps,
           g2_W1, g2_b1, g2_W2, g2_b2, g2_gamma, g2_beta, g2_eps,
           g3_W1, g3_b1, g3_W2, g3_b2, g3_gamma, g3_beta, g3_eps,
           m1_W, m2_W, att_W, tn_W, tn_Wb, tn_bias, fc_W, fc_b, sc_W, sc_b):
    f32 = jnp.float32
    packed = jnp.concatenate([
        (edge_index_1[0] << 14) | edge_index_1[1],
        ((edge_index_2[0] + _N) << 14) | edge_index_2[1]])
    row = lambda v: v.reshape(1, -1).astype(f32)
    sca = lambda v: v.reshape(1, 1).astype(f32)

    layer_p = [
        (g1_eps, g1_W1, g1_b1, g1_W2, g1_b2, g1_gamma, g1_beta),
        (g2_eps, g2_W1, g2_b1, g2_W2, g2_b2, g2_gamma, g2_beta),
        (g3_eps, g3_W1, g3_b1, g3_W2, g3_b2, g3_gamma, g3_beta),
    ]

    xh = [jnp.concatenate([features_1, features_2], axis=0)]
    colsum_a = None
    for li in (0, 1, 2):
        eps, W1, b1, W2, b2, gamma, beta = layer_p[li]
        aggs = _segsum(len(xh))(packed, *xh)
        if len(xh) == 1:
            aggs = (aggs,) if not isinstance(aggs, (list, tuple)) else aggs
        t, mom = _gin_mm[li](sca(eps), *xh, *aggs, W1, row(b1), W2, row(b2))
        outs = _norms[li](t, mom, row(gamma), row(beta))
        if li == 2:
            a, colsum_a = outs
            xh = [a]
        else:
            xh = list(outs)

    a = xh[0]
    match = _match_k(colsum_a.reshape(2, 64), m1_W, m2_W).reshape(2, 1, 64)
    colsum_x = _abs_colsum_k(a, match)
    p = _attention_k(a, match, colsum_x, att_W)
    score2, ged2 = _head_k(p.reshape(2, 64), tn_W.reshape(64, 64 * 16),
                           tn_Wb.T, tn_bias.reshape(1, 16),
                           fc_W, row(fc_b), sc_W, sca(sc_b), sca(avg_v))
    return score2.reshape(-1), ged2.reshape(-1)
